# baseline (speedup 1.0000x reference)
import functools

import jax
import jax.numpy as jnp
from jax import lax
from jax.experimental import pallas as pl
from jax.experimental.pallas import tpu as pltpu
from jax.experimental.pallas import tpu_sc as plsc

_NC = 2
_NS = 16
_NW = _NC * _NS
_LANES = 16

_B = 512
_G = _B // 128


def _enc_body(x_ref, w_ref, b_ref, o_ref):
    o_ref[...] = (
        jnp.dot(x_ref[...], w_ref[...], preferred_element_type=jnp.float32)
        + b_ref[...]
    )


def _leaky(x, s):
    return jnp.where(x >= 0, x, s * x)


def _dense_body(g_ref, xe_ref, wmu_ref, bmu_ref, wlv_ref, blv_ref,
                ws1_ref, bs1_ref, ws2_ref, bs2_ref, ws3_ref, bs3_ref,
                wa1_ref, ba1_ref, wa2_ref, ba2_ref,
                mu_ref, lv_ref, slog_ref, ssamp_ref, sstruct_ref):
    h = jnp.maximum(
        jnp.dot(g_ref[...], xe_ref[...], preferred_element_type=jnp.float32),
        0.0,
    )
    mu = jnp.dot(h, wmu_ref[...], preferred_element_type=jnp.float32) + bmu_ref[...]
    lv = jnp.dot(h, wlv_ref[...], preferred_element_type=jnp.float32) + blv_ref[...]
    l1 = _leaky(jnp.dot(mu, ws1_ref[...], preferred_element_type=jnp.float32)
                + bs1_ref[...], 0.2)
    l2 = _leaky(jnp.dot(l1, ws2_ref[...], preferred_element_type=jnp.float32)
                + bs2_ref[...], 0.2)
    slog = jnp.dot(l2, ws3_ref[...], preferred_element_type=jnp.float32) + bs3_ref[...]
    m = jnp.max(slog, axis=-1, keepdims=True)
    e = jnp.exp(slog - m)
    ssamp = e / jnp.sum(e, axis=-1, keepdims=True)
    a = _leaky(jnp.dot(ssamp, wa1_ref[...], preferred_element_type=jnp.float32)
               + ba1_ref[...], 0.01)
    sstruct = jnp.dot(a, wa2_ref[...], preferred_element_type=jnp.float32) + ba2_ref[...]
    mu_ref[...] = mu
    lv_ref[...] = lv
    slog_ref[...] = slog
    ssamp_ref[...] = ssamp
    sstruct_ref[...] = sstruct


def _fin_body(acc_ref, wg_ref, wf_ref, bf_ref, o_ref, *, n):
    wgf = jnp.dot(wg_ref[...], wf_ref[...], preferred_element_type=jnp.float32)
    p = jnp.concatenate([wgf, wgf], axis=0)
    xh = lax.dot_general(acc_ref[...], p, (((0,), (0,)), ((), ())),
                         preferred_element_type=jnp.float32)
    o_ref[...] = xh[:n] + bf_ref[...]


def _make_edge_kernel(n, np_pad, e, sdim):
    chunks = e // _B
    lanes = lambda: jax.lax.broadcasted_iota(jnp.int32, (_LANES,), 0)
    zeros16 = lambda: jnp.zeros((_LANES,), jnp.int32)
    ones16 = lambda: jnp.full((_LANES,), 1, jnp.int32)

    mesh = plsc.VectorSubcoreMesh(core_axis_name="c", subcore_axis_name="s",
                                  num_cores=_NC, num_subcores=_NS)

    @functools.partial(
        pl.kernel,
        out_type=[
            jax.ShapeDtypeStruct((e,), jnp.float32),
            jax.ShapeDtypeStruct((e,), jnp.float32),
            jax.ShapeDtypeStruct((_NC * 2, np_pad), jnp.float32),
        ],
        mesh=mesh,
        compiler_params=pltpu.CompilerParams(needs_layout_passes=False,
                                             use_tc_tiling_on_sc=False),
        scratch_types=[
            pltpu.VMEM((2, _G, 128), jnp.int32),
            pltpu.VMEM((2, _G, 128), jnp.int32),
            pltpu.VMEM((2 * _B, sdim), jnp.float32),
            pltpu.VMEM((2 * _B, sdim), jnp.float32),
            pltpu.VMEM((_B,), jnp.float32),
            pltpu.VMEM((_B,), jnp.float32),
            pltpu.VMEM((_G, 128), jnp.float32),
            pltpu.VMEM((_G, 128), jnp.float32),
            pltpu.VMEM((_G, 128), jnp.float32),
            pltpu.VMEM((_G, 128), jnp.float32),
            pltpu.VMEM((_G, 128), jnp.int32),
            pltpu.VMEM((_G, 128), jnp.int32),
            pltpu.VMEM((n,), jnp.float32),
            pltpu.VMEM((n,), jnp.float32),
            pltpu.VMEM_SHARED((np_pad,), jnp.float32),
            pltpu.VMEM_SHARED((np_pad,), jnp.float32),
            pltpu.SemaphoreType.DMA,
            pltpu.SemaphoreType.DMA,
            pltpu.SemaphoreType.DMA,
            pltpu.SemaphoreType.DMA,
            pltpu.SemaphoreType.DMA,
            pltpu.SemaphoreType.DMA,
        ],
    )
    def edge_kernel(sstruct_hbm, ssamp_hbm, pos_hbm, neg_hbm, zeros_hbm,
                    pos_out, neg_out, acc_out,
                    idxA, idxB, rowsA, rowsB, lbufA, lbufB, t0A, t1A, t0B,
                    t1B, sidxA, sidxB, stab0, stab1, accsh0, accsh1,
                    semgA, semgB, semiA, semiB, semo, semsc):
        c = lax.axis_index("c")
        s = lax.axis_index("s")
        w = s * _NC + c

        pltpu.sync_copy(ssamp_hbm.at[0], stab0)
        pltpu.sync_copy(ssamp_hbm.at[1], stab1)
        rows_per = np_pad // _NS
        r0 = s * rows_per
        pltpu.sync_copy(zeros_hbm.at[pl.ds(r0, rows_per)],
                        accsh0.at[pl.ds(r0, rows_per)])
        pltpu.sync_copy(zeros_hbm.at[pl.ds(r0, rows_per)],
                        accsh1.at[pl.ds(r0, rows_per)])
        plsc.subcore_barrier()

        n_chunks = chunks // _NW + jnp.where(w < chunks % _NW, 1, 0)
        max_chunks = (chunks + _NW - 1) // _NW
        n_pairs = (max_chunks + 1) // 2

        def fire_idx(ei_hbm, ci, idx, semi):
            row0 = (ci * _NW + w) * _G
            return pltpu.async_copy(ei_hbm.at[:, pl.ds(row0, _G)], idx, semi)

        def fire_gathers(idx, rows, semg):
            cps = []
            for j in range(_G):
                cps.append(pltpu.async_copy(
                    sstruct_hbm.at[idx.at[0, j]],
                    rows.at[pl.ds(j * 128, 128)], semg))
                cps.append(pltpu.async_copy(
                    sstruct_hbm.at[idx.at[1, j]],
                    rows.at[pl.ds(_B + j * 128, 128)], semg))
            return cps

        def drain(idx, rows, semg):
            for cp in fire_gathers(idx, rows, semg):
                cp.wait()

        _Q = 4

        def quad_dot(q, rows):
            evecs = [(q * _Q + gi) * 16 + lanes() for gi in range(_Q)]
            accs = [jnp.zeros((_LANES,), jnp.float32) for _ in range(_Q)]
            for k in range(sdim):
                kv = jnp.full((_LANES,), k, jnp.int32)
                svs = [plsc.load_gather(rows, [evecs[gi], kv])
                       for gi in range(_Q)]
                dvs = [plsc.load_gather(rows, [_B + evecs[gi], kv])
                       for gi in range(_Q)]
                for gi in range(_Q):
                    accs[gi] = accs[gi] + svs[gi] * dvs[gi]
            return accs

        def compute_chunk(ci, idx, rows, lbuf, t0, t1, sidx, out_hbm,
                          is_pos):
            def pos_quad(q):
                accs = quad_dot(q, rows)
                for gi in range(_Q):
                    g = q * _Q + gi
                    acc = accs[gi]
                    lbuf[pl.ds(g * 16, 16)] = acc
                    jv = g // 8
                    rr = (g % 8) * 16
                    srcids = idx[0, jv, pl.ds(rr, 16)]
                    s0 = plsc.load_gather(stab0, [srcids])
                    s1 = plsc.load_gather(stab1, [srcids])
                    t0[jv, pl.ds(rr, 16)] = acc * s0
                    t1[jv, pl.ds(rr, 16)] = acc * s1
                    sidx[jv, pl.ds(rr, 16)] = idx[1, jv, pl.ds(rr, 16)]

            def neg_quad(q):
                accs = quad_dot(q, rows)
                for gi in range(_Q):
                    g = q * _Q + gi
                    lbuf[pl.ds(g * 16, 16)] = accs[gi]

            plsc.parallel_loop(0, _B // 16 // _Q)(
                pos_quad if is_pos else neg_quad)
            eb = (ci * _NW + w) * _B
            pltpu.async_copy(lbuf, out_hbm.at[pl.ds(eb, _B)], semo)
            if is_pos:
                for j in range(_G):
                    pltpu.async_copy(t0.at[j], accsh0.at[sidx.at[j]], semsc,
                                     add=True)
                    pltpu.async_copy(t1.at[j], accsh1.at[sidx.at[j]], semsc,
                                     add=True)

        def wait_scatters(t0, t1, sidx):
            for j in range(_G):
                pltpu.make_async_copy(
                    t0.at[j], accsh0.at[sidx.at[j]], semsc).wait()
                pltpu.make_async_copy(
                    t1.at[j], accsh1.at[sidx.at[j]], semsc).wait()

        def wait_gathers(idx, rows, semg):
            for j in range(_G):
                pltpu.make_async_copy(
                    sstruct_hbm.at[idx.at[0, j]],
                    rows.at[pl.ds(j * 128, 128)], semg).wait()
                pltpu.make_async_copy(
                    sstruct_hbm.at[idx.at[1, j]],
                    rows.at[pl.ds(_B + j * 128, 128)], semg).wait()

        def run_set(ei_hbm, out_hbm, is_pos):
            @pl.when(n_chunks > 0)
            def _():
                fire_idx(ei_hbm, 0, idxA, semiA).wait()
                fire_gathers(idxA, rowsA, semgA)
            @pl.when(n_chunks > 1)
            def _():
                fire_idx(ei_hbm, 1, idxB, semiB)

            def phase(ci, idx, rows, lbuf, t0, t1, sidx, semg, oidx, orows,
                      osemg, osemi, csemi):
                @pl.when(ci < n_chunks)
                def _():
                    wait_gathers(idx, rows, semg)
                    @pl.when(ci + 1 < n_chunks)
                    def _():
                        row1 = ((ci + 1) * _NW + w) * _G
                        pltpu.make_async_copy(
                            ei_hbm.at[:, pl.ds(row1, _G)], oidx, osemi).wait()
                        fire_gathers(oidx, orows, osemg)
                    @pl.when(ci >= 2)
                    def _():
                        pltpu.make_async_copy(
                            lbuf, out_hbm.at[pl.ds(0, _B)], semo).wait()
                        if is_pos:
                            wait_scatters(t0, t1, sidx)
                    compute_chunk(ci, idx, rows, lbuf, t0, t1, sidx, out_hbm,
                                  is_pos)
                    @pl.when(ci + 2 < n_chunks)
                    def _():
                        fire_idx(ei_hbm, ci + 2, idx, csemi)

            def pair(pi, _):
                phase(2 * pi, idxA, rowsA, lbufA, t0A, t1A, sidxA, semgA,
                      idxB, rowsB, semgB, semiB, semiA)
                phase(2 * pi + 1, idxB, rowsB, lbufB, t0B, t1B, sidxB, semgB,
                      idxA, rowsA, semgA, semiA, semiB)
                return _

            lax.fori_loop(0, n_pairs, pair, 0)
            @pl.when(n_chunks >= 2)
            def _():
                pltpu.make_async_copy(
                    lbufA, out_hbm.at[pl.ds(0, _B)], semo).wait()
                if is_pos:
                    wait_scatters(t0A, t1A, sidxA)
            @pl.when(n_chunks >= 1)
            def _():
                pltpu.make_async_copy(
                    lbufA, out_hbm.at[pl.ds(0, _B)], semo).wait()
                if is_pos:
                    wait_scatters(t0B, t1B, sidxB)

        run_set(pos_hbm, pos_out, True)
        plsc.subcore_barrier()
        pltpu.sync_copy(accsh0.at[pl.ds(r0, rows_per)],
                        acc_out.at[c * 2, pl.ds(r0, rows_per)])
        pltpu.sync_copy(accsh1.at[pl.ds(r0, rows_per)],
                        acc_out.at[c * 2 + 1, pl.ds(r0, rows_per)])
        run_set(neg_hbm, neg_out, False)

    return edge_kernel


def kernel(g, x, pos_edge_index, neg_edge_index, W_enc, b_enc, W_mu, b_mu,
           W_lv, b_lv, W_s1, b_s1, W_s2, b_s2, W_s3, b_s3, W_a1, b_a1,
           W_a2, b_a2, W_g, W_f, b_f):
    n, d = x.shape
    h_dim = W_enc.shape[1]
    l_dim = W_mu.shape[1]
    e = pos_edge_index.shape[1]
    sdim = W_a2.shape[1]
    np_pad = ((n + 16 * _NS - 1) // (16 * _NS)) * (16 * _NS)

    xe = pl.pallas_call(
        _enc_body,
        out_shape=jax.ShapeDtypeStruct((n, h_dim), jnp.float32),
    )(x, W_enc, b_enc.reshape(1, h_dim))

    bm = 400
    n_blocks = n // bm
    full = lambda a: pl.BlockSpec(a.shape, lambda i: (0,) * a.ndim)
    w2 = [W_mu, b_mu.reshape(1, -1), W_lv, b_lv.reshape(1, -1),
          W_s1, b_s1.reshape(1, -1), W_s2, b_s2.reshape(1, -1),
          W_s3, b_s3.reshape(1, -1), W_a1, b_a1.reshape(1, -1),
          W_a2, b_a2.reshape(1, -1)]
    mu, logvar, s_logits, s_sample, s_struct = pl.pallas_call(
        _dense_body,
        grid=(n_blocks,),
        in_specs=[
            pl.BlockSpec((bm, n), lambda i: (i, 0)),
            pl.BlockSpec((n, h_dim), lambda i: (0, 0)),
        ] + [full(a) for a in w2],
        out_specs=[
            pl.BlockSpec((bm, l_dim), lambda i: (i, 0)),
            pl.BlockSpec((bm, l_dim), lambda i: (i, 0)),
            pl.BlockSpec((bm, 2), lambda i: (i, 0)),
            pl.BlockSpec((bm, 2), lambda i: (i, 0)),
            pl.BlockSpec((bm, sdim), lambda i: (i, 0)),
        ],
        out_shape=[
            jax.ShapeDtypeStruct((n, l_dim), jnp.float32),
            jax.ShapeDtypeStruct((n, l_dim), jnp.float32),
            jax.ShapeDtypeStruct((n, 2), jnp.float32),
            jax.ShapeDtypeStruct((n, 2), jnp.float32),
            jax.ShapeDtypeStruct((n, sdim), jnp.float32),
        ],
    )(g, xe, *w2)

    pos2 = pos_edge_index.astype(jnp.int32).reshape(2, e // 128, 128)
    neg2 = neg_edge_index.astype(jnp.int32).reshape(2, e // 128, 128)
    zeros_hbm = jnp.zeros((np_pad,), jnp.float32)

    edge_kernel = _make_edge_kernel(n, np_pad, e, sdim)
    pos_logits, neg_logits, acc = edge_kernel(
        s_struct, s_sample.T, pos2, neg2, zeros_hbm)

    x_hat = pl.pallas_call(
        functools.partial(_fin_body, n=n),
        out_shape=jax.ShapeDtypeStruct((n, d), jnp.float32),
    )(acc, W_g, W_f, b_f.reshape(1, d))

    return (x_hat, pos_logits, neg_logits, s_logits, mu, logvar, mu)


# speedup vs baseline: 2.0577x; 2.0577x over previous
import functools

import jax
import jax.numpy as jnp
from jax import lax
from jax.experimental import pallas as pl
from jax.experimental.pallas import tpu as pltpu
from jax.experimental.pallas import tpu_sc as plsc

_NC = 2
_NS = 16
_NW = _NC * _NS
_LANES = 16

_B = 512
_G = _B // 128


def _enc_body(x_ref, w_ref, b_ref, o_ref):
    o_ref[...] = (
        jnp.dot(x_ref[...], w_ref[...], preferred_element_type=jnp.float32)
        + b_ref[...]
    )


def _leaky(x, s):
    return jnp.where(x >= 0, x, s * x)


def _dense_body(g_ref, xe_ref, wmu_ref, bmu_ref, wlv_ref, blv_ref,
                ws1_ref, bs1_ref, ws2_ref, bs2_ref, ws3_ref, bs3_ref,
                wa1_ref, ba1_ref, wa2_ref, ba2_ref,
                mu_ref, lv_ref, slog_ref, ssamp_ref, sstruct_ref):
    h = jnp.maximum(
        jnp.dot(g_ref[...], xe_ref[...], preferred_element_type=jnp.float32),
        0.0,
    )
    mu = jnp.dot(h, wmu_ref[...], preferred_element_type=jnp.float32) + bmu_ref[...]
    lv = jnp.dot(h, wlv_ref[...], preferred_element_type=jnp.float32) + blv_ref[...]
    l1 = _leaky(jnp.dot(mu, ws1_ref[...], preferred_element_type=jnp.float32)
                + bs1_ref[...], 0.2)
    l2 = _leaky(jnp.dot(l1, ws2_ref[...], preferred_element_type=jnp.float32)
                + bs2_ref[...], 0.2)
    slog = jnp.dot(l2, ws3_ref[...], preferred_element_type=jnp.float32) + bs3_ref[...]
    m = jnp.max(slog, axis=-1, keepdims=True)
    e = jnp.exp(slog - m)
    ssamp = e / jnp.sum(e, axis=-1, keepdims=True)
    a = _leaky(jnp.dot(ssamp, wa1_ref[...], preferred_element_type=jnp.float32)
               + ba1_ref[...], 0.01)
    sstruct = jnp.dot(a, wa2_ref[...], preferred_element_type=jnp.float32) + ba2_ref[...]
    mu_ref[...] = mu
    lv_ref[...] = lv
    slog_ref[...] = slog
    ssamp_ref[...] = ssamp
    sstruct_ref[...] = sstruct


def _fin_body(acc_ref, wg_ref, wf_ref, bf_ref, o_ref, *, n):
    wgf = jnp.dot(wg_ref[...], wf_ref[...], preferred_element_type=jnp.float32)
    p = jnp.concatenate([wgf, wgf], axis=0)
    xh = lax.dot_general(acc_ref[...], p, (((0,), (0,)), ((), ())),
                         preferred_element_type=jnp.float32)
    o_ref[...] = xh[:n] + bf_ref[...]


def _make_edge_kernel(n, np_pad, e, sdim):
    chunks = e // _B
    lanes = lambda: jax.lax.broadcasted_iota(jnp.int32, (_LANES,), 0)
    zeros16 = lambda: jnp.zeros((_LANES,), jnp.int32)
    ones16 = lambda: jnp.full((_LANES,), 1, jnp.int32)

    mesh = plsc.VectorSubcoreMesh(core_axis_name="c", subcore_axis_name="s",
                                  num_cores=_NC, num_subcores=_NS)

    @functools.partial(
        pl.kernel,
        out_type=[
            jax.ShapeDtypeStruct((e,), jnp.float32),
            jax.ShapeDtypeStruct((e,), jnp.float32),
            jax.ShapeDtypeStruct((_NC * 2, np_pad), jnp.float32),
        ],
        mesh=mesh,
        compiler_params=pltpu.CompilerParams(needs_layout_passes=False,
                                             use_tc_tiling_on_sc=False),
        scratch_types=[
            pltpu.VMEM((2, _G, 128), jnp.int32),
            pltpu.VMEM((2, _G, 128), jnp.int32),
            pltpu.VMEM((2 * _B, sdim), jnp.float32),
            pltpu.VMEM((2 * _B, sdim), jnp.float32),
            pltpu.VMEM((_B,), jnp.float32),
            pltpu.VMEM((_B,), jnp.float32),
            pltpu.VMEM((_G, 128), jnp.float32),
            pltpu.VMEM((_G, 128), jnp.float32),
            pltpu.VMEM((_G, 128), jnp.float32),
            pltpu.VMEM((_G, 128), jnp.float32),
            pltpu.VMEM((_G, 128), jnp.int32),
            pltpu.VMEM((_G, 128), jnp.int32),
            pltpu.VMEM((n,), jnp.float32),
            pltpu.VMEM((n,), jnp.float32),
            pltpu.VMEM_SHARED((np_pad,), jnp.float32),
            pltpu.VMEM_SHARED((np_pad,), jnp.float32),
            pltpu.SemaphoreType.DMA,
            pltpu.SemaphoreType.DMA,
            pltpu.SemaphoreType.DMA,
            pltpu.SemaphoreType.DMA,
            pltpu.SemaphoreType.DMA,
            pltpu.SemaphoreType.DMA,
        ],
    )
    def edge_kernel(sstruct_hbm, ssamp_hbm, pos_hbm, neg_hbm, zeros_hbm,
                    pos_out, neg_out, acc_out,
                    idxA, idxB, rowsA, rowsB, lbufA, lbufB, t0A, t1A, t0B,
                    t1B, sidxA, sidxB, stab0, stab1, accsh0, accsh1,
                    semgA, semgB, semiA, semiB, semo, semsc):
        c = lax.axis_index("c")
        s = lax.axis_index("s")
        w = s * _NC + c

        pltpu.sync_copy(ssamp_hbm.at[0], stab0)
        pltpu.sync_copy(ssamp_hbm.at[1], stab1)
        rows_per = np_pad // _NS
        r0 = s * rows_per
        pltpu.sync_copy(zeros_hbm.at[pl.ds(r0, rows_per)],
                        accsh0.at[pl.ds(r0, rows_per)])
        pltpu.sync_copy(zeros_hbm.at[pl.ds(r0, rows_per)],
                        accsh1.at[pl.ds(r0, rows_per)])
        plsc.subcore_barrier()

        n_chunks = chunks // _NW + jnp.where(w < chunks % _NW, 1, 0)
        max_chunks = (chunks + _NW - 1) // _NW
        n_pairs = (max_chunks + 1) // 2

        def fire_idx(ei_hbm, ci, idx, semi):
            row0 = (ci * _NW + w) * _G
            return pltpu.async_copy(ei_hbm.at[:, pl.ds(row0, _G)], idx, semi)

        def fire_gathers(idx, rows, semg):
            cps = []
            for j in range(_G):
                cps.append(pltpu.async_copy(
                    sstruct_hbm.at[idx.at[0, j]],
                    rows.at[pl.ds(j * 128, 128)], semg))
                cps.append(pltpu.async_copy(
                    sstruct_hbm.at[idx.at[1, j]],
                    rows.at[pl.ds(_B + j * 128, 128)], semg))
            return cps

        def drain(idx, rows, semg):
            for cp in fire_gathers(idx, rows, semg):
                cp.wait()

        _Q = 4

        def quad_dot(q, rows):
            evecs = [(q * _Q + gi) * 16 + lanes() for gi in range(_Q)]
            accs = [jnp.zeros((_LANES,), jnp.float32) for _ in range(_Q)]
            for k in range(sdim):
                kv = (lanes() + k) & (sdim - 1)
                svs = [plsc.load_gather(rows, [evecs[gi], kv])
                       for gi in range(_Q)]
                dvs = [plsc.load_gather(rows, [_B + evecs[gi], kv])
                       for gi in range(_Q)]
                for gi in range(_Q):
                    accs[gi] = accs[gi] + svs[gi] * dvs[gi]
            return accs

        def compute_chunk(ci, idx, rows, lbuf, t0, t1, sidx, out_hbm,
                          is_pos):
            def pos_quad(q):
                accs = quad_dot(q, rows)
                for gi in range(_Q):
                    g = q * _Q + gi
                    acc = accs[gi]
                    lbuf[pl.ds(g * 16, 16)] = acc
                    jv = g // 8
                    rr = (g % 8) * 16
                    srcids = idx[0, jv, pl.ds(rr, 16)]
                    s0 = plsc.load_gather(stab0, [srcids])
                    s1 = plsc.load_gather(stab1, [srcids])
                    t0[jv, pl.ds(rr, 16)] = acc * s0
                    t1[jv, pl.ds(rr, 16)] = acc * s1
                    sidx[jv, pl.ds(rr, 16)] = idx[1, jv, pl.ds(rr, 16)]

            def neg_quad(q):
                accs = quad_dot(q, rows)
                for gi in range(_Q):
                    g = q * _Q + gi
                    lbuf[pl.ds(g * 16, 16)] = accs[gi]

            plsc.parallel_loop(0, _B // 16 // _Q)(
                pos_quad if is_pos else neg_quad)
            eb = (ci * _NW + w) * _B
            pltpu.async_copy(lbuf, out_hbm.at[pl.ds(eb, _B)], semo)
            if is_pos:
                for j in range(_G):
                    pltpu.async_copy(t0.at[j], accsh0.at[sidx.at[j]], semsc,
                                     add=True)
                    pltpu.async_copy(t1.at[j], accsh1.at[sidx.at[j]], semsc,
                                     add=True)

        def wait_scatters(t0, t1, sidx):
            for j in range(_G):
                pltpu.make_async_copy(
                    t0.at[j], accsh0.at[sidx.at[j]], semsc).wait()
                pltpu.make_async_copy(
                    t1.at[j], accsh1.at[sidx.at[j]], semsc).wait()

        def wait_gathers(idx, rows, semg):
            for j in range(_G):
                pltpu.make_async_copy(
                    sstruct_hbm.at[idx.at[0, j]],
                    rows.at[pl.ds(j * 128, 128)], semg).wait()
                pltpu.make_async_copy(
                    sstruct_hbm.at[idx.at[1, j]],
                    rows.at[pl.ds(_B + j * 128, 128)], semg).wait()

        def run_set(ei_hbm, out_hbm, is_pos):
            @pl.when(n_chunks > 0)
            def _():
                fire_idx(ei_hbm, 0, idxA, semiA).wait()
                fire_gathers(idxA, rowsA, semgA)
            @pl.when(n_chunks > 1)
            def _():
                fire_idx(ei_hbm, 1, idxB, semiB)

            def phase(ci, idx, rows, lbuf, t0, t1, sidx, semg, oidx, orows,
                      osemg, osemi, csemi):
                @pl.when(ci < n_chunks)
                def _():
                    wait_gathers(idx, rows, semg)
                    @pl.when(ci + 1 < n_chunks)
                    def _():
                        row1 = ((ci + 1) * _NW + w) * _G
                        pltpu.make_async_copy(
                            ei_hbm.at[:, pl.ds(row1, _G)], oidx, osemi).wait()
                        fire_gathers(oidx, orows, osemg)
                    @pl.when(ci >= 2)
                    def _():
                        pltpu.make_async_copy(
                            lbuf, out_hbm.at[pl.ds(0, _B)], semo).wait()
                        if is_pos:
                            wait_scatters(t0, t1, sidx)
                    compute_chunk(ci, idx, rows, lbuf, t0, t1, sidx, out_hbm,
                                  is_pos)
                    @pl.when(ci + 2 < n_chunks)
                    def _():
                        fire_idx(ei_hbm, ci + 2, idx, csemi)

            def pair(pi, _):
                phase(2 * pi, idxA, rowsA, lbufA, t0A, t1A, sidxA, semgA,
                      idxB, rowsB, semgB, semiB, semiA)
                phase(2 * pi + 1, idxB, rowsB, lbufB, t0B, t1B, sidxB, semgB,
                      idxA, rowsA, semgA, semiA, semiB)
                return _

            lax.fori_loop(0, n_pairs, pair, 0)
            @pl.when(n_chunks >= 2)
            def _():
                pltpu.make_async_copy(
                    lbufA, out_hbm.at[pl.ds(0, _B)], semo).wait()
                if is_pos:
                    wait_scatters(t0A, t1A, sidxA)
            @pl.when(n_chunks >= 1)
            def _():
                pltpu.make_async_copy(
                    lbufA, out_hbm.at[pl.ds(0, _B)], semo).wait()
                if is_pos:
                    wait_scatters(t0B, t1B, sidxB)

        run_set(pos_hbm, pos_out, True)
        plsc.subcore_barrier()
        pltpu.sync_copy(accsh0.at[pl.ds(r0, rows_per)],
                        acc_out.at[c * 2, pl.ds(r0, rows_per)])
        pltpu.sync_copy(accsh1.at[pl.ds(r0, rows_per)],
                        acc_out.at[c * 2 + 1, pl.ds(r0, rows_per)])
        run_set(neg_hbm, neg_out, False)

    return edge_kernel


def kernel(g, x, pos_edge_index, neg_edge_index, W_enc, b_enc, W_mu, b_mu,
           W_lv, b_lv, W_s1, b_s1, W_s2, b_s2, W_s3, b_s3, W_a1, b_a1,
           W_a2, b_a2, W_g, W_f, b_f):
    n, d = x.shape
    h_dim = W_enc.shape[1]
    l_dim = W_mu.shape[1]
    e = pos_edge_index.shape[1]
    sdim = W_a2.shape[1]
    np_pad = ((n + 16 * _NS - 1) // (16 * _NS)) * (16 * _NS)

    xe = pl.pallas_call(
        _enc_body,
        out_shape=jax.ShapeDtypeStruct((n, h_dim), jnp.float32),
    )(x, W_enc, b_enc.reshape(1, h_dim))

    bm = 400
    n_blocks = n // bm
    full = lambda a: pl.BlockSpec(a.shape, lambda i: (0,) * a.ndim)
    w2 = [W_mu, b_mu.reshape(1, -1), W_lv, b_lv.reshape(1, -1),
          W_s1, b_s1.reshape(1, -1), W_s2, b_s2.reshape(1, -1),
          W_s3, b_s3.reshape(1, -1), W_a1, b_a1.reshape(1, -1),
          W_a2, b_a2.reshape(1, -1)]
    mu, logvar, s_logits, s_sample, s_struct = pl.pallas_call(
        _dense_body,
        grid=(n_blocks,),
        in_specs=[
            pl.BlockSpec((bm, n), lambda i: (i, 0)),
            pl.BlockSpec((n, h_dim), lambda i: (0, 0)),
        ] + [full(a) for a in w2],
        out_specs=[
            pl.BlockSpec((bm, l_dim), lambda i: (i, 0)),
            pl.BlockSpec((bm, l_dim), lambda i: (i, 0)),
            pl.BlockSpec((bm, 2), lambda i: (i, 0)),
            pl.BlockSpec((bm, 2), lambda i: (i, 0)),
            pl.BlockSpec((bm, sdim), lambda i: (i, 0)),
        ],
        out_shape=[
            jax.ShapeDtypeStruct((n, l_dim), jnp.float32),
            jax.ShapeDtypeStruct((n, l_dim), jnp.float32),
            jax.ShapeDtypeStruct((n, 2), jnp.float32),
            jax.ShapeDtypeStruct((n, 2), jnp.float32),
            jax.ShapeDtypeStruct((n, sdim), jnp.float32),
        ],
    )(g, xe, *w2)

    pos2 = pos_edge_index.astype(jnp.int32).reshape(2, e // 128, 128)
    neg2 = neg_edge_index.astype(jnp.int32).reshape(2, e // 128, 128)
    zeros_hbm = jnp.zeros((np_pad,), jnp.float32)

    edge_kernel = _make_edge_kernel(n, np_pad, e, sdim)
    pos_logits, neg_logits, acc = edge_kernel(
        s_struct, s_sample.T, pos2, neg2, zeros_hbm)

    x_hat = pl.pallas_call(
        functools.partial(_fin_body, n=n),
        out_shape=jax.ShapeDtypeStruct((n, d), jnp.float32),
    )(acc, W_g, W_f, b_f.reshape(1, d))

    return (x_hat, pos_logits, neg_logits, s_logits, mu, logvar, mu)
